# Initial kernel scaffold; baseline (speedup 1.0000x reference)
import jax
import jax.numpy as jnp
from jax.experimental import pallas as pl


def kernel(lead_times, pe):
    raise NotImplementedError("write your pallas kernel here")


# speedup vs baseline: 1.6790x; 1.6790x over previous
import functools

import jax
import jax.numpy as jnp
from jax import lax
from jax.experimental import pallas as pl
from jax.experimental.pallas import tpu as pltpu
from jax.experimental.pallas import tpu_sc as plsc

EMBEDDING_DIM = 128
MAX_LEAD_TIME = 72
BATCH = 16384
LANES = 16
IDX_CHUNK = 128


def kernel(lead_times, pe):
    info = plsc.get_sparse_core_info()
    num_cores, num_subcores = info.num_cores, info.num_subcores
    num_workers = num_cores * num_subcores
    b_per_w = BATCH // num_workers
    n_chunks = b_per_w // IDX_CHUNK

    mesh = plsc.VectorSubcoreMesh(core_axis_name="c", subcore_axis_name="s")

    @functools.partial(
        pl.kernel,
        mesh=mesh,
        out_type=jax.ShapeDtypeStruct((BATCH, EMBEDDING_DIM), jnp.float32),
        scratch_types=[
            pltpu.VMEM((b_per_w,), jnp.int32),
            pltpu.VMEM((b_per_w, EMBEDDING_DIM), jnp.float32),
            pltpu.SemaphoreType.DMA,
        ],
    )
    def emb_kernel(lt_hbm, pe_hbm, out_hbm, idx_v, rows_v, sem):
        wid = lax.axis_index("s") * num_cores + lax.axis_index("c")
        base = wid * b_per_w
        pltpu.sync_copy(lt_hbm.at[pl.ds(base, b_per_w)], idx_v)
        for i in range(b_per_w // LANES):
            v = idx_v[pl.ds(i * LANES, LANES)]
            idx_v[pl.ds(i * LANES, LANES)] = jnp.minimum(
                jnp.maximum(v, 0), MAX_LEAD_TIME
            )
        copies = []
        for c in range(n_chunks):
            copies.append(
                pltpu.async_copy(
                    pe_hbm.at[idx_v.at[pl.ds(c * IDX_CHUNK, IDX_CHUNK)]],
                    rows_v.at[pl.ds(c * IDX_CHUNK, IDX_CHUNK)],
                    sem,
                )
            )
        for cp in copies:
            cp.wait()
        pltpu.sync_copy(rows_v, out_hbm.at[pl.ds(base, b_per_w)])

    return emb_kernel(lead_times.astype(jnp.int32), pe)
